# Initial kernel scaffold; baseline (speedup 1.0000x reference)
import functools

import jax
import jax.numpy as jnp
from jax import lax
from jax.experimental import pallas as pl
from jax.experimental.pallas import tpu as pltpu
from jax.experimental.pallas import tpu_sc as plsc

N = 10000
E = 320000
D = 128
H = 128
DE = 16
G = 64

NC = 2
NS = 16
NW = NC * NS
EW = E // NW
C = 80
NCHUNK = EW // C
RPS = N // NS
ZR = 125


def _sc_scatter_body(h_hbm, ea_hbm, src_hbm, dst_hbm, out_hbm,
                     aggr_sh, srcv, dstv, eav, rows, zbuf, sem):
    c = lax.axis_index("c")
    s = lax.axis_index("s")
    wid = c * NS + s

    def zrow(i, carry):
        for j in range(8):
            zbuf[i, pl.ds(j * 16, 16)] = jnp.zeros((16,), jnp.float32)
        return carry
    lax.fori_loop(0, ZR, zrow, 0)
    for k in range(RPS // ZR):
        pltpu.sync_copy(zbuf, aggr_sh.at[pl.ds(s * RPS + k * ZR, ZR)])
    plsc.subcore_barrier()

    def chunk(t, carry):
        base = wid * EW + t * C
        pltpu.sync_copy(src_hbm.at[pl.ds(base, C)], srcv)
        pltpu.sync_copy(dst_hbm.at[pl.ds(base, C)], dstv)
        cp = pltpu.async_copy(h_hbm.at[srcv], rows, sem)
        pltpu.sync_copy(ea_hbm.at[pl.ds(base, C)], eav)
        cp.wait()

        def erow(i, icarry):
            for j in range(8):
                sl = pl.ds(j * 16, 16)
                rows[i, sl] = jnp.maximum(rows[i, sl] + eav[i, sl], 0.0)
            return icarry
        lax.fori_loop(0, C, erow, 0)

        pltpu.sync_copy(rows, aggr_sh.at[dstv], add=True)
        return carry
    lax.fori_loop(0, NCHUNK, chunk, 0)
    plsc.subcore_barrier()

    pltpu.sync_copy(aggr_sh.at[pl.ds(s * RPS, RPS)],
                    out_hbm.at[c, pl.ds(s * RPS, RPS)])


@jax.jit
def _sc_scatter(h, ea, src, dst):
    mesh = plsc.VectorSubcoreMesh(core_axis_name="c", subcore_axis_name="s",
                                  num_cores=NC, num_subcores=NS)
    return pl.kernel(
        _sc_scatter_body,
        out_type=jax.ShapeDtypeStruct((NC, N, D), jnp.float32),
        mesh=mesh,
        scratch_types=[
            pltpu.VMEM_SHARED((N, D), jnp.float32),
            pltpu.VMEM((C,), jnp.int32),
            pltpu.VMEM((C,), jnp.int32),
            pltpu.VMEM((C, D), jnp.float32),
            pltpu.VMEM((C, D), jnp.float32),
            pltpu.VMEM((ZR, D), jnp.float32),
            pltpu.SemaphoreType.DMA,
        ],
    )(h, ea, src, dst)


def _edge_proj_body(ea_ref, w0_ref, b0_ref, w1_ref, b1_ref, o0_ref, o1_ref):
    a = ea_ref[...]
    o0_ref[...] = jnp.dot(a, w0_ref[...],
                          preferred_element_type=jnp.float32) + b0_ref[...]
    o1_ref[...] = jnp.dot(a, w1_ref[...],
                          preferred_element_type=jnp.float32) + b1_ref[...]


@jax.jit
def _edge_proj(edge_attr, ep_w0, ep_b0, ep_w1, ep_b1):
    BE = 4000
    grid = (E // BE,)
    return pl.pallas_call(
        _edge_proj_body,
        grid=grid,
        in_specs=[
            pl.BlockSpec((BE, DE), lambda i: (i, 0)),
            pl.BlockSpec((DE, D), lambda i: (0, 0)),
            pl.BlockSpec((1, D), lambda i: (0, 0)),
            pl.BlockSpec((DE, D), lambda i: (0, 0)),
            pl.BlockSpec((1, D), lambda i: (0, 0)),
        ],
        out_specs=[
            pl.BlockSpec((BE, D), lambda i: (i, 0)),
            pl.BlockSpec((BE, D), lambda i: (i, 0)),
        ],
        out_shape=[
            jax.ShapeDtypeStruct((E, D), jnp.float32),
            jax.ShapeDtypeStruct((E, D), jnp.float32),
        ],
    )(edge_attr, ep_w0, ep_b0.reshape(1, D), ep_w1, ep_b1.reshape(1, D))


def _mlp_bn(h, aggr0, aggr1, w1, b1, w2, b2, gamma, beta):
    z = h + aggr0 + aggr1
    y = jnp.dot(jnp.maximum(jnp.dot(z, w1, preferred_element_type=jnp.float32)
                            + b1, 0.0),
                w2, preferred_element_type=jnp.float32) + b2
    mu = jnp.mean(y, axis=0, keepdims=True)
    var = jnp.mean((y - mu) * (y - mu), axis=0, keepdims=True)
    yn = (y - mu) * lax.rsqrt(var + 1e-5) * gamma + beta
    return jnp.maximum(yn, 0.0)


def _node_body(h_ref, p_ref, w1_ref, b1_ref, w2_ref, b2_ref, g_ref, be_ref,
               out_ref):
    out_ref[...] = _mlp_bn(h_ref[...], p_ref[0], p_ref[1],
                           w1_ref[...], b1_ref[...], w2_ref[...], b2_ref[...],
                           g_ref[...], be_ref[...])


@jax.jit
def _node(h, partials, w1, b1, w2, b2, gamma, beta):
    return pl.pallas_call(
        _node_body,
        out_shape=jax.ShapeDtypeStruct((N, H), jnp.float32),
    )(h, partials, w1, b1.reshape(1, H), w2, b2.reshape(1, H),
      gamma.reshape(1, H), beta.reshape(1, H))


def _node_pool_body(h_ref, p_ref, w1_ref, b1_ref, w2_ref, b2_ref, g_ref,
                    be_ref, batch_ref, l1w_ref, l1b_ref, l2w_ref, l2b_ref,
                    out_ref, gm_ref):
    h2 = _mlp_bn(h_ref[...], p_ref[0], p_ref[1],
                 w1_ref[...], b1_ref[...], w2_ref[...], b2_ref[...],
                 g_ref[...], be_ref[...])
    b = batch_ref[...]
    for g in range(G):
        m = jnp.where(b == g, h2, -jnp.float32(3.0e38))
        gm_ref[pl.ds(g, 1), :] = jnp.max(m, axis=0, keepdims=True)
    gm = gm_ref[...]
    y = jnp.maximum(jnp.dot(gm, l1w_ref[...],
                            preferred_element_type=jnp.float32)
                    + l1b_ref[...], 0.0)
    out_ref[...] = jnp.dot(y, l2w_ref[...],
                           preferred_element_type=jnp.float32) + l2b_ref[...]


@jax.jit
def _node_pool(h, partials, w1, b1, w2, b2, gamma, beta, batch2,
               l1w, l1b, l2w, l2b):
    return pl.pallas_call(
        _node_pool_body,
        out_shape=jax.ShapeDtypeStruct((G, 2), jnp.float32),
        scratch_shapes=[pltpu.VMEM((G, H), jnp.float32)],
    )(h, partials, w1, b1.reshape(1, H), w2, b2.reshape(1, H),
      gamma.reshape(1, H), beta.reshape(1, H), batch2,
      l1w, l1b.reshape(1, H), l2w, l2b.reshape(1, 2))


def kernel(x, edge_index, edge_attr, batch, ep_w0, ep_b0, w1_0, b1_0, w2_0,
           b2_0, g0, be0, ep_w1, ep_b1, w1_1, b1_1, w2_1, b2_1, g1, be1,
           lin1_w, lin1_b, lin2_w, lin2_b):
    src = edge_index[0]
    dst = edge_index[1]
    batch2 = batch.reshape(N, 1)

    ea0, ea1 = _edge_proj(edge_attr, ep_w0, ep_b0, ep_w1, ep_b1)
    p0 = _sc_scatter(x, ea0, src, dst)
    h1 = _node(x, p0, w1_0, b1_0, w2_0, b2_0, g0, be0)
    p1 = _sc_scatter(h1, ea1, src, dst)
    return _node_pool(h1, p1, w1_1, b1_1, w2_1, b2_1, g1, be1, batch2,
                      lin1_w, lin1_b, lin2_w, lin2_b)


# speedup vs baseline: 2.7293x; 2.7293x over previous
import functools

import jax
import jax.numpy as jnp
from jax import lax
from jax.experimental import pallas as pl
from jax.experimental.pallas import tpu as pltpu
from jax.experimental.pallas import tpu_sc as plsc

N = 10000
E = 320000
D = 128
H = 128
DE = 16
G = 64

NC = 2
NS = 16
NW = NC * NS
EW = E // NW
C = 80
NCHUNK = EW // C
ZR = 200
NZCH = N // ZR


def _sc_scatter_body(h_hbm, ea_hbm, src_hbm, dst_hbm, out_hbm,
                     aggr_sh, srcv, dstv, eav, rows, zbuf, sem):
    c = lax.axis_index("c")
    s = lax.axis_index("s")
    wid = c * NS + s

    def zrow(i, carry):
        for j in range(8):
            zbuf[i, pl.ds(j * 16, 16)] = jnp.zeros((16,), jnp.float32)
        return carry
    lax.fori_loop(0, ZR, zrow, 0)

    nz = (NZCH - s + NS - 1) // NS

    def zchunk(t, carry):
        q = s + t * NS
        pltpu.sync_copy(zbuf, aggr_sh.at[pl.ds(q * ZR, ZR)])
        return carry
    lax.fori_loop(0, nz, zchunk, 0)
    plsc.subcore_barrier()

    def chunk(t, carry):
        base = wid * EW + t * C
        pltpu.sync_copy(src_hbm.at[pl.ds(base, C)], srcv)
        pltpu.sync_copy(dst_hbm.at[pl.ds(base, C)], dstv)
        cp = pltpu.async_copy(h_hbm.at[srcv], rows, sem)
        pltpu.sync_copy(ea_hbm.at[pl.ds(base, C)], eav)
        cp.wait()

        def erow(i, icarry):
            for j in range(8):
                sl = pl.ds(j * 16, 16)
                rows[i, sl] = jnp.maximum(rows[i, sl] + eav[i, sl], 0.0)
            return icarry
        lax.fori_loop(0, C, erow, 0)

        pltpu.sync_copy(rows, aggr_sh.at[dstv], add=True)
        return carry
    lax.fori_loop(0, NCHUNK, chunk, 0)
    plsc.subcore_barrier()

    def ochunk(t, carry):
        q = s + t * NS
        pltpu.sync_copy(aggr_sh.at[pl.ds(q * ZR, ZR)],
                        out_hbm.at[c, pl.ds(q * ZR, ZR)])
        return carry
    lax.fori_loop(0, nz, ochunk, 0)


@jax.jit
def _sc_scatter(h, ea, src, dst):
    mesh = plsc.VectorSubcoreMesh(core_axis_name="c", subcore_axis_name="s",
                                  num_cores=NC, num_subcores=NS)
    return pl.kernel(
        _sc_scatter_body,
        out_type=jax.ShapeDtypeStruct((NC, N, D), jnp.float32),
        mesh=mesh,
        scratch_types=[
            pltpu.VMEM_SHARED((N, D), jnp.float32),
            pltpu.VMEM((C,), jnp.int32),
            pltpu.VMEM((C,), jnp.int32),
            pltpu.VMEM((C, D), jnp.float32),
            pltpu.VMEM((C, D), jnp.float32),
            pltpu.VMEM((ZR, D), jnp.float32),
            pltpu.SemaphoreType.DMA,
        ],
    )(h, ea, src, dst)


def _edge_proj_body(ea_ref, w0_ref, b0_ref, w1_ref, b1_ref, o0_ref, o1_ref):
    a = ea_ref[...]
    o0_ref[...] = jnp.dot(a, w0_ref[...],
                          preferred_element_type=jnp.float32) + b0_ref[...]
    o1_ref[...] = jnp.dot(a, w1_ref[...],
                          preferred_element_type=jnp.float32) + b1_ref[...]


@jax.jit
def _edge_proj(edge_attr, ep_w0, ep_b0, ep_w1, ep_b1):
    BE = 4000
    grid = (E // BE,)
    return pl.pallas_call(
        _edge_proj_body,
        grid=grid,
        in_specs=[
            pl.BlockSpec((BE, DE), lambda i: (i, 0)),
            pl.BlockSpec((DE, D), lambda i: (0, 0)),
            pl.BlockSpec((1, D), lambda i: (0, 0)),
            pl.BlockSpec((DE, D), lambda i: (0, 0)),
            pl.BlockSpec((1, D), lambda i: (0, 0)),
        ],
        out_specs=[
            pl.BlockSpec((BE, D), lambda i: (i, 0)),
            pl.BlockSpec((BE, D), lambda i: (i, 0)),
        ],
        out_shape=[
            jax.ShapeDtypeStruct((E, D), jnp.float32),
            jax.ShapeDtypeStruct((E, D), jnp.float32),
        ],
    )(edge_attr, ep_w0, ep_b0.reshape(1, D), ep_w1, ep_b1.reshape(1, D))


def _mlp_bn(h, aggr0, aggr1, w1, b1, w2, b2, gamma, beta):
    z = h + aggr0 + aggr1
    y = jnp.dot(jnp.maximum(jnp.dot(z, w1, preferred_element_type=jnp.float32)
                            + b1, 0.0),
                w2, preferred_element_type=jnp.float32) + b2
    mu = jnp.mean(y, axis=0, keepdims=True)
    var = jnp.mean((y - mu) * (y - mu), axis=0, keepdims=True)
    yn = (y - mu) * lax.rsqrt(var + 1e-5) * gamma + beta
    return jnp.maximum(yn, 0.0)


def _node_body(h_ref, p_ref, w1_ref, b1_ref, w2_ref, b2_ref, g_ref, be_ref,
               out_ref):
    out_ref[...] = _mlp_bn(h_ref[...], p_ref[0], p_ref[1],
                           w1_ref[...], b1_ref[...], w2_ref[...], b2_ref[...],
                           g_ref[...], be_ref[...])


@jax.jit
def _node(h, partials, w1, b1, w2, b2, gamma, beta):
    return pl.pallas_call(
        _node_body,
        out_shape=jax.ShapeDtypeStruct((N, H), jnp.float32),
    )(h, partials, w1, b1.reshape(1, H), w2, b2.reshape(1, H),
      gamma.reshape(1, H), beta.reshape(1, H))


def _node_pool_body(h_ref, p_ref, w1_ref, b1_ref, w2_ref, b2_ref, g_ref,
                    be_ref, batch_ref, l1w_ref, l1b_ref, l2w_ref, l2b_ref,
                    out_ref, gm_ref):
    h2 = _mlp_bn(h_ref[...], p_ref[0], p_ref[1],
                 w1_ref[...], b1_ref[...], w2_ref[...], b2_ref[...],
                 g_ref[...], be_ref[...])
    b = batch_ref[...]
    for g in range(G):
        m = jnp.where(b == g, h2, -jnp.inf)
        gm_ref[pl.ds(g, 1), :] = jnp.max(m, axis=0, keepdims=True)
    gm = gm_ref[...]
    y = jnp.maximum(jnp.dot(gm, l1w_ref[...],
                            preferred_element_type=jnp.float32)
                    + l1b_ref[...], 0.0)
    out_ref[...] = jnp.dot(y, l2w_ref[...],
                           preferred_element_type=jnp.float32) + l2b_ref[...]


@jax.jit
def _node_pool(h, partials, w1, b1, w2, b2, gamma, beta, batch2,
               l1w, l1b, l2w, l2b):
    return pl.pallas_call(
        _node_pool_body,
        out_shape=jax.ShapeDtypeStruct((G, 2), jnp.float32),
        scratch_shapes=[pltpu.VMEM((G, H), jnp.float32)],
    )(h, partials, w1, b1.reshape(1, H), w2, b2.reshape(1, H),
      gamma.reshape(1, H), beta.reshape(1, H), batch2,
      l1w, l1b.reshape(1, H), l2w, l2b.reshape(1, 2))


def kernel(x, edge_index, edge_attr, batch, ep_w0, ep_b0, w1_0, b1_0, w2_0,
           b2_0, g0, be0, ep_w1, ep_b1, w1_1, b1_1, w2_1, b2_1, g1, be1,
           lin1_w, lin1_b, lin2_w, lin2_b):
    src = edge_index[0]
    dst = edge_index[1]
    batch2 = batch.reshape(N, 1)

    ea0, ea1 = _edge_proj(edge_attr, ep_w0, ep_b0, ep_w1, ep_b1)
    p0 = _sc_scatter(x, ea0, src, dst)
    h1 = _node(x, p0, w1_0, b1_0, w2_0, b2_0, g0, be0)
    p1 = _sc_scatter(h1, ea1, src, dst)
    return _node_pool(h1, p1, w1_1, b1_1, w2_1, b2_1, g1, be1, batch2,
                      lin1_w, lin1_b, lin2_w, lin2_b)
